# baseline (speedup 1.0000x reference)
import functools

import jax
import jax.numpy as jnp
from jax import lax
from jax.experimental import pallas as pl
from jax.experimental.pallas import tpu as pltpu
from jax.experimental.pallas import tpu_sc as plsc

TOPK = 2
BT = 256


def _router_body(x_ref, wr_ref, out_ref):
    logits = lax.dot_general(
        x_ref[...], wr_ref[...], (((1,), (0,)), ((), ())),
        preferred_element_type=jnp.float32)
    e = logits.shape[1]
    iota = lax.broadcasted_iota(jnp.int32, logits.shape, 1)
    v1 = jnp.max(logits, axis=1, keepdims=True)
    i1 = jnp.min(jnp.where(logits == v1, iota, e), axis=1, keepdims=True)
    masked = jnp.where(iota == i1, -jnp.inf, logits)
    v2 = jnp.max(masked, axis=1, keepdims=True)
    i2 = jnp.min(jnp.where(masked == v2, iota, e), axis=1, keepdims=True)
    t = jnp.exp(v2 - v1)
    w1 = 1.0 / (1.0 + t)
    w2 = t / (1.0 + t)
    z = jnp.zeros_like(w1)
    out_ref[...] = jnp.concatenate(
        [i1.astype(jnp.float32), i2.astype(jnp.float32), w1, w2, z, z, z, z],
        axis=1)


def _router(xf, wr):
    T, D = xf.shape
    BTR = 512
    return pl.pallas_call(
        _router_body,
        grid=(T // BTR,),
        in_specs=[
            pl.BlockSpec((BTR, D), lambda b: (b, 0)),
            pl.BlockSpec((D, wr.shape[1]), lambda b: (0, 0)),
        ],
        out_specs=pl.BlockSpec((BTR, 8), lambda b: (b, 0)),
        out_shape=jax.ShapeDtypeStruct((T, 8), jnp.float32),
    )(xf, wr)


def _sc_dispatch(xw, pslot1, pslot2, ns_max):
    T, W = xw.shape
    mesh = plsc.VectorSubcoreMesh(core_axis_name="c", subcore_axis_name="s")
    nw = 32
    per_w = T // nw
    chunk = 64
    n_chunks = per_w // chunk

    @functools.partial(
        pl.kernel,
        out_type=jax.ShapeDtypeStruct((ns_max, W), jnp.float32),
        mesh=mesh,
        scratch_types=[
            pltpu.VMEM((chunk,), jnp.int32),
            pltpu.VMEM((chunk,), jnp.int32),
            pltpu.VMEM((chunk, W), jnp.float32),
            pltpu.SemaphoreType.DMA,
            pltpu.SemaphoreType.DMA,
        ],
    )
    def k(x_hbm, p1_hbm, p2_hbm, out_hbm, i1_v, i2_v, rows_v, sem1, sem2):
        wid = lax.axis_index("s") * 2 + lax.axis_index("c")
        base = wid * per_w

        def body(ci, carry):
            t0 = base + ci * chunk
            pltpu.sync_copy(p1_hbm.at[pl.ds(t0, chunk)], i1_v)
            pltpu.sync_copy(p2_hbm.at[pl.ds(t0, chunk)], i2_v)
            pltpu.sync_copy(x_hbm.at[pl.ds(t0, chunk), :], rows_v)
            c1 = pltpu.async_copy(rows_v, out_hbm.at[i1_v], sem1)
            c2 = pltpu.async_copy(rows_v, out_hbm.at[i2_v], sem2)
            c1.wait()
            c2.wait()
            return carry

        lax.fori_loop(0, n_chunks, body, 0)

    return k(xw, pslot1, pslot2)


def _gate_up_body(be_ref, na_ref, xs_ref, w1_ref, w3_ref, g_ref,
                  w1c_ref, w3c_ref):
    b = pl.program_id(1)
    new_w = jnp.logical_or(b == 0, be_ref[b] != be_ref[jnp.maximum(b - 1, 0)])

    @pl.when(jnp.logical_and(new_w, b < na_ref[0]))
    def _():
        w1c_ref[...] = w1_ref[0].astype(jnp.bfloat16)
        w3c_ref[...] = w3_ref[0].astype(jnp.bfloat16)

    @pl.when(b < na_ref[0])
    def _():
        xb = xs_ref[...].astype(jnp.bfloat16)
        h = lax.dot_general(xb, w1c_ref[...], (((1,), (0,)), ((), ())),
                            preferred_element_type=jnp.float32)
        u = lax.dot_general(xb, w3c_ref[...], (((1,), (0,)), ((), ())),
                            preferred_element_type=jnp.float32)
        g = (h * jax.nn.sigmoid(h)) * u
        g_ref[...] = g.astype(jnp.bfloat16)


def _down_body(be_ref, na_ref, g_ref, w2_ref, ys_ref, w2c_ref):
    b = pl.program_id(0)
    new_w = jnp.logical_or(b == 0, be_ref[b] != be_ref[jnp.maximum(b - 1, 0)])

    @pl.when(jnp.logical_and(new_w, b < na_ref[0]))
    def _():
        w2c_ref[...] = w2_ref[0].astype(jnp.bfloat16)

    @pl.when(b < na_ref[0])
    def _():
        ys_ref[...] = lax.dot_general(
            g_ref[...], w2c_ref[...], (((1,), (0,)), ((), ())),
            preferred_element_type=jnp.float32)


def _grouped_swiglu(xs, w1b, w3b, w2b, block_expert, num_active):
    ns_max, D = xs.shape
    E, _, FF = w1b.shape
    nb = ns_max // BT

    FT = FF // 2
    g = pl.pallas_call(
        _gate_up_body,
        grid_spec=pltpu.PrefetchScalarGridSpec(
            num_scalar_prefetch=2,
            grid=(FF // FT, nb),
            in_specs=[
                pl.BlockSpec((BT, D), lambda ft, b, be, na: (b, 0)),
                pl.BlockSpec((1, D, FT), lambda ft, b, be, na: (be[b], 0, ft)),
                pl.BlockSpec((1, D, FT), lambda ft, b, be, na: (be[b], 0, ft)),
            ],
            out_specs=pl.BlockSpec((BT, FT), lambda ft, b, be, na: (b, ft)),
            scratch_shapes=[pltpu.VMEM((D, FT), jnp.bfloat16),
                            pltpu.VMEM((D, FT), jnp.bfloat16)],
        ),
        out_shape=jax.ShapeDtypeStruct((ns_max, FF), jnp.bfloat16),
    )(block_expert, num_active, xs, w1b, w3b)

    ys = pl.pallas_call(
        _down_body,
        grid_spec=pltpu.PrefetchScalarGridSpec(
            num_scalar_prefetch=2,
            grid=(nb,),
            in_specs=[
                pl.BlockSpec((BT, FF), lambda b, be, na: (b, 0)),
                pl.BlockSpec((1, FF, D), lambda b, be, na: (be[b], 0, 0)),
            ],
            out_specs=pl.BlockSpec((BT, D), lambda b, be, na: (b, 0)),
            scratch_shapes=[pltpu.VMEM((FF, D), jnp.bfloat16)],
        ),
        out_shape=jax.ShapeDtypeStruct((ns_max, D), jnp.float32),
    )(block_expert, num_active, g, w2b)
    return ys


def _sc_combine(ys, pos1, pos2, w1, w2):
    ns_max, D = ys.shape
    T = pos1.shape[0]
    mesh = plsc.VectorSubcoreMesh(core_axis_name="c", subcore_axis_name="s")
    nw = 32
    per_w = T // nw
    chunk = 32
    n_chunks = per_w // chunk
    n16 = D // 16

    @functools.partial(
        pl.kernel,
        out_type=jax.ShapeDtypeStruct((T, D), jnp.float32),
        mesh=mesh,
        scratch_types=[
            pltpu.VMEM((chunk,), jnp.int32),
            pltpu.VMEM((chunk,), jnp.int32),
            pltpu.VMEM((chunk,), jnp.float32),
            pltpu.VMEM((chunk,), jnp.float32),
            pltpu.VMEM((chunk, D), jnp.float32),
            pltpu.VMEM((chunk, D), jnp.float32),
            pltpu.SemaphoreType.DMA,
            pltpu.SemaphoreType.DMA,
        ],
        compiler_params=pltpu.CompilerParams(needs_layout_passes=False),
    )
    def k(ys_hbm, p1_hbm, p2_hbm, w1_hbm, w2_hbm, out_hbm,
          i1_v, i2_v, w1_v, w2_v, a_v, b_v, sem1, sem2):
        wid = lax.axis_index("s") * 2 + lax.axis_index("c")
        base = wid * per_w

        def body(ci, carry):
            t0 = base + ci * chunk
            pltpu.sync_copy(p1_hbm.at[pl.ds(t0, chunk)], i1_v)
            pltpu.sync_copy(p2_hbm.at[pl.ds(t0, chunk)], i2_v)
            pltpu.sync_copy(w1_hbm.at[pl.ds(t0, chunk)], w1_v)
            pltpu.sync_copy(w2_hbm.at[pl.ds(t0, chunk)], w2_v)
            c1 = pltpu.async_copy(ys_hbm.at[i1_v], a_v, sem1)
            c2 = pltpu.async_copy(ys_hbm.at[i2_v], b_v, sem2)
            c1.wait()
            c2.wait()

            def row(r, rc):
                ridx = jnp.full((16,), r, jnp.int32)
                wa = plsc.load_gather(w1_v, [ridx])
                wb = plsc.load_gather(w2_v, [ridx])
                for j in range(n16):
                    sl = pl.ds(j * 16, 16)
                    a_v[r, sl] = a_v[r, sl] * wa + b_v[r, sl] * wb
                return rc

            lax.fori_loop(0, chunk, row, 0)
            pltpu.sync_copy(a_v, out_hbm.at[pl.ds(t0, chunk), :])
            return carry

        lax.fori_loop(0, n_chunks, body, 0)

    return k(ys, pos1, pos2, w1, w2)


def _dispatch_metadata(i1, i2, E, nb):
    T = i1.shape[0]
    tk = TOPK * T
    experts = jnp.concatenate([i1, i2])

    ohf = (experts[:, None] == jnp.arange(E)[None, :]).astype(jnp.float32)
    C = 128
    nch = tk // C
    oh3 = ohf.reshape(nch, C, E)
    tri_c = (jnp.arange(C)[:, None] >= jnp.arange(C)[None, :]
             ).astype(jnp.float32)
    within = jnp.einsum('jk,cke->cje', tri_c, oh3,
                        precision=lax.Precision.HIGHEST)
    chunk_tot = within[:, -1, :]
    tri_n = (jnp.arange(nch)[:, None] > jnp.arange(nch)[None, :]
             ).astype(jnp.float32)
    chunk_pre = jnp.einsum('dc,ce->de', tri_n, chunk_tot,
                           precision=lax.Precision.HIGHEST)
    csum3 = within + chunk_pre[:, None, :]
    counts = (chunk_pre[-1] + chunk_tot[-1]).astype(jnp.int32)
    padded = ((counts + BT - 1) // BT) * BT
    tri_e = (jnp.arange(E)[:, None] >= jnp.arange(E)[None, :]
             ).astype(jnp.float32)
    ends_padded = jnp.einsum('fe,e->f', tri_e, padded.astype(jnp.float32),
                             precision=lax.Precision.HIGHEST
                             ).astype(jnp.int32)
    starts_padded = ends_padded - padded

    rank3 = jnp.sum(oh3 * csum3, axis=2) - 1.0
    base3 = jnp.sum(oh3 * starts_padded.astype(jnp.float32)[None, None, :],
                    axis=2)
    pslot = (base3 + rank3).reshape(tk).astype(jnp.int32)
    pos1, pos2 = pslot[:T], pslot[T:]

    block_expert = jnp.minimum(
        jnp.sum((jnp.arange(nb)[:, None] * BT >= ends_padded[None, :]
                 ).astype(jnp.int32), axis=1),
        E - 1).astype(jnp.int32)
    num_active = (jnp.sum(padded) // BT).astype(jnp.int32)[None]
    return pos1, pos2, block_expert, num_active


def kernel(x, Wr, W1, W2, W3):
    b, s, d = x.shape
    T = b * s
    E = Wr.shape[1]
    FF = W1.shape[2]
    nb = T * TOPK // BT + E
    ns_max = nb * BT

    xf = x.reshape(T, d)
    r = _router(xf, Wr)
    i1 = r[:, 0].astype(jnp.int32)
    i2 = r[:, 1].astype(jnp.int32)
    pos1, pos2, block_expert, num_active = _dispatch_metadata(i1, i2, E, nb)

    xs = _sc_dispatch(xf, pos1, pos2, ns_max)
    ys = _grouped_swiglu(xs, W1, W3, W2, block_expert, num_active)
    out = _sc_combine(ys, pos1, pos2, r[:, 2], r[:, 3])
    return out.reshape(b, s, d)


# speedup vs baseline: 1.0242x; 1.0242x over previous
import functools

import jax
import jax.numpy as jnp
from jax import lax
from jax.experimental import pallas as pl
from jax.experimental.pallas import tpu as pltpu
from jax.experimental.pallas import tpu_sc as plsc

TOPK = 2
BT = 512


def _router_body(x_ref, wr_ref, out_ref):
    logits = lax.dot_general(
        x_ref[...], wr_ref[...], (((1,), (0,)), ((), ())),
        preferred_element_type=jnp.float32)
    e = logits.shape[1]
    iota = lax.broadcasted_iota(jnp.int32, logits.shape, 1)
    v1 = jnp.max(logits, axis=1, keepdims=True)
    i1 = jnp.min(jnp.where(logits == v1, iota, e), axis=1, keepdims=True)
    masked = jnp.where(iota == i1, -jnp.inf, logits)
    v2 = jnp.max(masked, axis=1, keepdims=True)
    i2 = jnp.min(jnp.where(masked == v2, iota, e), axis=1, keepdims=True)
    t = jnp.exp(v2 - v1)
    w1 = 1.0 / (1.0 + t)
    w2 = t / (1.0 + t)
    z = jnp.zeros_like(w1)
    out_ref[...] = jnp.concatenate(
        [i1.astype(jnp.float32), i2.astype(jnp.float32), w1, w2, z, z, z, z],
        axis=1)


def _router(xf, wr):
    T, D = xf.shape
    BTR = 512
    return pl.pallas_call(
        _router_body,
        grid=(T // BTR,),
        in_specs=[
            pl.BlockSpec((BTR, D), lambda b: (b, 0)),
            pl.BlockSpec((D, wr.shape[1]), lambda b: (0, 0)),
        ],
        out_specs=pl.BlockSpec((BTR, 8), lambda b: (b, 0)),
        out_shape=jax.ShapeDtypeStruct((T, 8), jnp.float32),
    )(xf, wr)


def _sc_dispatch(xw, pslot1, pslot2, ns_max):
    T, W = xw.shape
    mesh = plsc.VectorSubcoreMesh(core_axis_name="c", subcore_axis_name="s")
    nw = 32
    per_w = T // nw
    chunk = 64
    n_chunks = per_w // chunk

    @functools.partial(
        pl.kernel,
        out_type=jax.ShapeDtypeStruct((ns_max, W), jnp.float32),
        mesh=mesh,
        scratch_types=[
            pltpu.VMEM((chunk,), jnp.int32),
            pltpu.VMEM((chunk,), jnp.int32),
            pltpu.VMEM((chunk, W), jnp.float32),
            pltpu.SemaphoreType.DMA,
            pltpu.SemaphoreType.DMA,
        ],
    )
    def k(x_hbm, p1_hbm, p2_hbm, out_hbm, i1_v, i2_v, rows_v, sem1, sem2):
        wid = lax.axis_index("s") * 2 + lax.axis_index("c")
        base = wid * per_w

        def body(ci, carry):
            t0 = base + ci * chunk
            pltpu.sync_copy(p1_hbm.at[pl.ds(t0, chunk)], i1_v)
            pltpu.sync_copy(p2_hbm.at[pl.ds(t0, chunk)], i2_v)
            pltpu.sync_copy(x_hbm.at[pl.ds(t0, chunk), :], rows_v)
            c1 = pltpu.async_copy(rows_v, out_hbm.at[i1_v], sem1)
            c2 = pltpu.async_copy(rows_v, out_hbm.at[i2_v], sem2)
            c1.wait()
            c2.wait()
            return carry

        lax.fori_loop(0, n_chunks, body, 0)

    return k(xw, pslot1, pslot2)


def _gate_up_body(be_ref, na_ref, xs_ref, w1_ref, w3_ref, g_ref,
                  w1c_ref, w3c_ref):
    b = pl.program_id(1)
    new_w = jnp.logical_or(b == 0, be_ref[b] != be_ref[jnp.maximum(b - 1, 0)])

    @pl.when(jnp.logical_and(new_w, b < na_ref[0]))
    def _():
        w1c_ref[...] = w1_ref[0].astype(jnp.bfloat16)
        w3c_ref[...] = w3_ref[0].astype(jnp.bfloat16)

    @pl.when(b < na_ref[0])
    def _():
        xb = xs_ref[...].astype(jnp.bfloat16)
        h = lax.dot_general(xb, w1c_ref[...], (((1,), (0,)), ((), ())),
                            preferred_element_type=jnp.float32)
        u = lax.dot_general(xb, w3c_ref[...], (((1,), (0,)), ((), ())),
                            preferred_element_type=jnp.float32)
        g = (h * jax.nn.sigmoid(h)) * u
        g_ref[...] = g.astype(jnp.bfloat16)


def _down_body(be_ref, na_ref, g_ref, w2_ref, ys_ref, w2c_ref):
    b = pl.program_id(0)
    new_w = jnp.logical_or(b == 0, be_ref[b] != be_ref[jnp.maximum(b - 1, 0)])

    @pl.when(jnp.logical_and(new_w, b < na_ref[0]))
    def _():
        w2c_ref[...] = w2_ref[0].astype(jnp.bfloat16)

    @pl.when(b < na_ref[0])
    def _():
        ys_ref[...] = lax.dot_general(
            g_ref[...], w2c_ref[...], (((1,), (0,)), ((), ())),
            preferred_element_type=jnp.float32)


def _grouped_swiglu(xs, w1b, w3b, w2b, block_expert, num_active):
    ns_max, D = xs.shape
    E, _, FF = w1b.shape
    nb = ns_max // BT

    FT = FF // 2
    g = pl.pallas_call(
        _gate_up_body,
        grid_spec=pltpu.PrefetchScalarGridSpec(
            num_scalar_prefetch=2,
            grid=(FF // FT, nb),
            in_specs=[
                pl.BlockSpec((BT, D), lambda ft, b, be, na: (b, 0)),
                pl.BlockSpec((1, D, FT), lambda ft, b, be, na: (be[b], 0, ft)),
                pl.BlockSpec((1, D, FT), lambda ft, b, be, na: (be[b], 0, ft)),
            ],
            out_specs=pl.BlockSpec((BT, FT), lambda ft, b, be, na: (b, ft)),
            scratch_shapes=[pltpu.VMEM((D, FT), jnp.bfloat16),
                            pltpu.VMEM((D, FT), jnp.bfloat16)],
        ),
        out_shape=jax.ShapeDtypeStruct((ns_max, FF), jnp.bfloat16),
    )(block_expert, num_active, xs, w1b, w3b)

    ys = pl.pallas_call(
        _down_body,
        grid_spec=pltpu.PrefetchScalarGridSpec(
            num_scalar_prefetch=2,
            grid=(nb,),
            in_specs=[
                pl.BlockSpec((BT, FF), lambda b, be, na: (b, 0)),
                pl.BlockSpec((1, FF, D), lambda b, be, na: (be[b], 0, 0)),
            ],
            out_specs=pl.BlockSpec((BT, D), lambda b, be, na: (b, 0)),
            scratch_shapes=[pltpu.VMEM((FF, D), jnp.bfloat16)],
        ),
        out_shape=jax.ShapeDtypeStruct((ns_max, D), jnp.float32),
    )(block_expert, num_active, g, w2b)
    return ys


def _sc_combine(ys, pos1, pos2, w1, w2):
    ns_max, D = ys.shape
    T = pos1.shape[0]
    mesh = plsc.VectorSubcoreMesh(core_axis_name="c", subcore_axis_name="s")
    nw = 32
    per_w = T // nw
    chunk = 32
    n_chunks = per_w // chunk
    n16 = D // 16

    @functools.partial(
        pl.kernel,
        out_type=jax.ShapeDtypeStruct((T, D), jnp.float32),
        mesh=mesh,
        scratch_types=[
            pltpu.VMEM((chunk,), jnp.int32),
            pltpu.VMEM((chunk,), jnp.int32),
            pltpu.VMEM((chunk,), jnp.float32),
            pltpu.VMEM((chunk,), jnp.float32),
            pltpu.VMEM((chunk, D), jnp.float32),
            pltpu.VMEM((chunk, D), jnp.float32),
            pltpu.SemaphoreType.DMA,
            pltpu.SemaphoreType.DMA,
        ],
        compiler_params=pltpu.CompilerParams(needs_layout_passes=False),
    )
    def k(ys_hbm, p1_hbm, p2_hbm, w1_hbm, w2_hbm, out_hbm,
          i1_v, i2_v, w1_v, w2_v, a_v, b_v, sem1, sem2):
        wid = lax.axis_index("s") * 2 + lax.axis_index("c")
        base = wid * per_w

        def body(ci, carry):
            t0 = base + ci * chunk
            pltpu.sync_copy(p1_hbm.at[pl.ds(t0, chunk)], i1_v)
            pltpu.sync_copy(p2_hbm.at[pl.ds(t0, chunk)], i2_v)
            pltpu.sync_copy(w1_hbm.at[pl.ds(t0, chunk)], w1_v)
            pltpu.sync_copy(w2_hbm.at[pl.ds(t0, chunk)], w2_v)
            c1 = pltpu.async_copy(ys_hbm.at[i1_v], a_v, sem1)
            c2 = pltpu.async_copy(ys_hbm.at[i2_v], b_v, sem2)
            c1.wait()
            c2.wait()

            def row(r, rc):
                ridx = jnp.full((16,), r, jnp.int32)
                wa = plsc.load_gather(w1_v, [ridx])
                wb = plsc.load_gather(w2_v, [ridx])
                for j in range(n16):
                    sl = pl.ds(j * 16, 16)
                    a_v[r, sl] = a_v[r, sl] * wa + b_v[r, sl] * wb
                return rc

            lax.fori_loop(0, chunk, row, 0)
            pltpu.sync_copy(a_v, out_hbm.at[pl.ds(t0, chunk), :])
            return carry

        lax.fori_loop(0, n_chunks, body, 0)

    return k(ys, pos1, pos2, w1, w2)


def _dispatch_metadata(i1, i2, E, nb):
    T = i1.shape[0]
    tk = TOPK * T
    experts = jnp.concatenate([i1, i2])

    ohf = (experts[:, None] == jnp.arange(E)[None, :]).astype(jnp.float32)
    C = 128
    nch = tk // C
    oh3 = ohf.reshape(nch, C, E)
    tri_c = (jnp.arange(C)[:, None] >= jnp.arange(C)[None, :]
             ).astype(jnp.float32)
    within = jnp.einsum('jk,cke->cje', tri_c, oh3,
                        precision=lax.Precision.HIGHEST)
    chunk_tot = within[:, -1, :]
    tri_n = (jnp.arange(nch)[:, None] > jnp.arange(nch)[None, :]
             ).astype(jnp.float32)
    chunk_pre = jnp.einsum('dc,ce->de', tri_n, chunk_tot,
                           precision=lax.Precision.HIGHEST)
    csum3 = within + chunk_pre[:, None, :]
    counts = (chunk_pre[-1] + chunk_tot[-1]).astype(jnp.int32)
    padded = ((counts + BT - 1) // BT) * BT
    tri_e = (jnp.arange(E)[:, None] >= jnp.arange(E)[None, :]
             ).astype(jnp.float32)
    ends_padded = jnp.einsum('fe,e->f', tri_e, padded.astype(jnp.float32),
                             precision=lax.Precision.HIGHEST
                             ).astype(jnp.int32)
    starts_padded = ends_padded - padded

    rank3 = jnp.sum(oh3 * csum3, axis=2) - 1.0
    base3 = jnp.sum(oh3 * starts_padded.astype(jnp.float32)[None, None, :],
                    axis=2)
    pslot = (base3 + rank3).reshape(tk).astype(jnp.int32)
    pos1, pos2 = pslot[:T], pslot[T:]

    block_expert = jnp.minimum(
        jnp.sum((jnp.arange(nb)[:, None] * BT >= ends_padded[None, :]
                 ).astype(jnp.int32), axis=1),
        E - 1).astype(jnp.int32)
    num_active = (jnp.sum(padded) // BT).astype(jnp.int32)[None]
    return pos1, pos2, block_expert, num_active


def kernel(x, Wr, W1, W2, W3):
    b, s, d = x.shape
    T = b * s
    E = Wr.shape[1]
    FF = W1.shape[2]
    nb = T * TOPK // BT + E
    ns_max = nb * BT

    xf = x.reshape(T, d)
    r = _router(xf, Wr)
    i1 = r[:, 0].astype(jnp.int32)
    i2 = r[:, 1].astype(jnp.int32)
    pos1, pos2, block_expert, num_active = _dispatch_metadata(i1, i2, E, nb)

    xs = _sc_dispatch(xf, pos1, pos2, ns_max)
    ys = _grouped_swiglu(xs, W1, W3, W2, block_expert, num_active)
    out = _sc_combine(ys, pos1, pos2, r[:, 2], r[:, 3])
    return out.reshape(b, s, d)
